# baseline (speedup 1.0000x reference)
import functools

import jax
import jax.numpy as jnp
from jax import lax
from jax.experimental import pallas as pl
from jax.experimental.pallas import tpu as pltpu
from jax.experimental.pallas import tpu_sc as plsc

N = 10000
E = 160000
FEAT = 14
DIM = 32
NOUT = 105
NG = 256
NSTEM = 20000
NJB = 10000

NP = 10240
EP = 163840
NC = 2
NS = 16
NW = NC * NS
CH = 256
ECHUNKS = EP // NW // CH
NSLC = NP // NS

STEM_P = 20480
JB_P = 10240
RLEN = STEM_P + 2 * JB_P
RCH = 256
RCHUNKS = RLEN // NW // RCH

_mesh = plsc.VectorSubcoreMesh(core_axis_name="c", subcore_axis_name="s")
_sc_params = pltpu.CompilerParams(use_tc_tiling_on_sc=False)


def _leaky(v):
    return jnp.where(v >= 0, v, 0.01 * v)


def _sc_gather_body(nchunks, chunk, per_worker, table, idx_hbm, out_hbm, idxv, rowsv, sem):
    cid = lax.axis_index("c")
    sid = lax.axis_index("s")
    wid = sid * NC + cid
    for ci in range(nchunks):
        off = wid * per_worker + ci * chunk
        pltpu.sync_copy(idx_hbm.at[pl.ds(off, chunk)], idxv)
        pltpu.async_copy(table.at[idxv], rowsv, sem).wait()
        pltpu.sync_copy(rowsv, out_hbm.at[pl.ds(off, chunk)])


def _make_sc_gather(total, nchunks, chunk):
    per_worker = total // NW
    return functools.partial(
        pl.kernel,
        out_type=jax.ShapeDtypeStruct((total, DIM), jnp.float32),
        mesh=_mesh,
        scratch_types=[
            pltpu.VMEM((chunk,), jnp.int32),
            pltpu.VMEM((chunk, DIM), jnp.float32),
            pltpu.SemaphoreType.DMA,
        ],
        compiler_params=_sc_params,
    )(functools.partial(_sc_gather_body, nchunks, chunk, per_worker))


_sc_gather_edges = _make_sc_gather(EP, ECHUNKS, CH)
_sc_gather_read = _make_sc_gather(RLEN, RCHUNKS, RCH)


def _sc_scatter_body(msg_hbm, dst_hbm, zeros_hbm, aggp_hbm, idxv, msgv, acc):
    cid = lax.axis_index("c")
    sid = lax.axis_index("s")
    pltpu.sync_copy(zeros_hbm.at[pl.ds(sid * NSLC, NSLC)], acc.at[pl.ds(sid * NSLC, NSLC)])
    plsc.subcore_barrier()
    half = EP // NC
    for ci in range(ECHUNKS):
        off = cid * half + sid * (half // NS) + ci * CH
        pltpu.sync_copy(dst_hbm.at[pl.ds(off, CH)], idxv)
        pltpu.sync_copy(msg_hbm.at[pl.ds(off, CH)], msgv)
        pltpu.sync_copy(msgv, acc.at[idxv], add=True)
    plsc.subcore_barrier()
    pltpu.sync_copy(acc.at[pl.ds(sid * NSLC, NSLC)], aggp_hbm.at[cid, pl.ds(sid * NSLC, NSLC)])


_sc_scatter = functools.partial(
    pl.kernel,
    out_type=jax.ShapeDtypeStruct((NC, NP, DIM), jnp.float32),
    mesh=_mesh,
    scratch_types=[
        pltpu.VMEM((CH,), jnp.int32),
        pltpu.VMEM((CH, DIM), jnp.float32),
        pltpu.VMEM_SHARED((NP, DIM), jnp.float32),
    ],
    compiler_params=_sc_params,
)(_sc_scatter_body)


def _sc_deg_body(dst_hbm, ones_hbm, zeros_hbm, degp_hbm, idxv, onesv, acc):
    cid = lax.axis_index("c")
    sid = lax.axis_index("s")
    pltpu.sync_copy(zeros_hbm.at[pl.ds(sid * NSLC, NSLC)], acc.at[pl.ds(sid * NSLC, NSLC)])
    pltpu.sync_copy(ones_hbm, onesv)
    plsc.subcore_barrier()
    half = EP // NC
    for ci in range(ECHUNKS):
        off = cid * half + sid * (half // NS) + ci * CH
        pltpu.sync_copy(dst_hbm.at[pl.ds(off, CH)], idxv)
        pltpu.sync_copy(onesv, acc.at[idxv], add=True)
    plsc.subcore_barrier()
    pltpu.sync_copy(acc.at[pl.ds(sid * NSLC, NSLC)], degp_hbm.at[cid, pl.ds(sid * NSLC, NSLC)])


_sc_deg = functools.partial(
    pl.kernel,
    out_type=jax.ShapeDtypeStruct((NC, NP, DIM), jnp.float32),
    mesh=_mesh,
    scratch_types=[
        pltpu.VMEM((CH,), jnp.int32),
        pltpu.VMEM((CH, DIM), jnp.float32),
        pltpu.VMEM_SHARED((NP, DIM), jnp.float32),
    ],
    compiler_params=_sc_params,
)(_sc_deg_body)


def _tc_init_body(xp_ref, w0_ref, b0_ref, degp_ref, s0_ref, dinv_ref):
    s0_ref[...] = _leaky(
        jnp.dot(xp_ref[...], w0_ref[...], preferred_element_type=jnp.float32) + b0_ref[...]
    )
    deg = jnp.maximum(degp_ref[0] + degp_ref[1], 1.0)
    dinv_ref[...] = 1.0 / deg


def _tc_init(xp, w0, b0r, degp):
    return pl.pallas_call(
        _tc_init_body,
        out_shape=(
            jax.ShapeDtypeStruct((NP, DIM), jnp.float32),
            jax.ShapeDtypeStruct((NP, DIM), jnp.float32),
        ),
    )(xp, w0, b0r, degp)


BE = 2048


def _tc_msg_body(ea_ref, xs_ref, wn1_ref, bn1_ref, tf_ref, bm_ref, msg_ref):
    z = _leaky(
        jnp.dot(ea_ref[...], wn1_ref[...], preferred_element_type=jnp.float32) + bn1_ref[...]
    )
    xs = xs_ref[...]
    u = (z[:, :, None] * xs[:, None, :]).reshape(BE, DIM * DIM)
    msg_ref[...] = (
        jnp.dot(u, tf_ref[...], preferred_element_type=jnp.float32)
        + jnp.dot(xs, bm_ref[...], preferred_element_type=jnp.float32)
    )


def _tc_msg(eap, xs, wn1, bn1r, tf, bm):
    nblk = EP // BE
    return pl.pallas_call(
        _tc_msg_body,
        grid=(nblk,),
        in_specs=[
            pl.BlockSpec((BE, 4), lambda i: (i, 0)),
            pl.BlockSpec((BE, DIM), lambda i: (i, 0)),
            pl.BlockSpec((4, DIM), lambda i: (0, 0)),
            pl.BlockSpec((1, DIM), lambda i: (0, 0)),
            pl.BlockSpec((DIM * DIM, DIM), lambda i: (0, 0)),
            pl.BlockSpec((DIM, DIM), lambda i: (0, 0)),
        ],
        out_specs=pl.BlockSpec((BE, DIM), lambda i: (i, 0)),
        out_shape=jax.ShapeDtypeStruct((EP, DIM), jnp.float32),
    )(eap, xs, wn1, bn1r, tf, bm)


def _tc_gru_body(s_ref, aggp_ref, dinv_ref, wroot_ref, broot_ref, wih_ref, bih_ref,
                 whh_ref, bhh_ref, out_ref):
    s = s_ref[...]
    agg = (aggp_ref[0] + aggp_ref[1]) * dinv_ref[...]
    m = _leaky(
        jnp.dot(s, wroot_ref[...], preferred_element_type=jnp.float32)
        + broot_ref[...] + agg
    )
    gi = jnp.dot(m, wih_ref[...], preferred_element_type=jnp.float32) + bih_ref[...]
    gh = jnp.dot(s, whh_ref[...], preferred_element_type=jnp.float32) + bhh_ref[...]
    r = jax.nn.sigmoid(gi[:, 0:DIM] + gh[:, 0:DIM])
    zz = jax.nn.sigmoid(gi[:, DIM:2 * DIM] + gh[:, DIM:2 * DIM])
    n = jnp.tanh(gi[:, 2 * DIM:3 * DIM] + r * gh[:, 2 * DIM:3 * DIM])
    out_ref[...] = (1.0 - zz) * n + zz * s


def _tc_gru(s, aggp, dinv, wroot, brootr, wih, bihr, whh, bhhr):
    return pl.pallas_call(
        _tc_gru_body,
        out_shape=jax.ShapeDtypeStruct((NP, DIM), jnp.float32),
    )(s, aggp, dinv, wroot, brootr, wih, bihr, whh, bhhr)


BS = 2048


def _tc_stem_body(g_ref, ws1_ref, bs1_ref, ws2_ref, bs2_ref, out_ref):
    act = _leaky(
        jnp.dot(g_ref[...], ws1_ref[...], preferred_element_type=jnp.float32) + bs1_ref[...]
    )
    out_ref[...] = jnp.dot(act, ws2_ref[...], preferred_element_type=jnp.float32) + bs2_ref[...]


def _tc_stem(gath, ws1, bs1r, ws2, bs2r):
    return pl.pallas_call(
        _tc_stem_body,
        grid=(STEM_P // BS,),
        in_specs=[
            pl.BlockSpec((BS, DIM), lambda i: (i, 0)),
            pl.BlockSpec((DIM, DIM), lambda i: (0, 0)),
            pl.BlockSpec((1, DIM), lambda i: (0, 0)),
            pl.BlockSpec((DIM, NOUT), lambda i: (0, 0)),
            pl.BlockSpec((1, NOUT), lambda i: (0, 0)),
        ],
        out_specs=pl.BlockSpec((BS, NOUT), lambda i: (i, 0)),
        out_shape=jax.ShapeDtypeStruct((STEM_P, NOUT), jnp.float32),
    )(gath, ws1, bs1r, ws2, bs2r)


def _tc_jb_body(g0_ref, g1_ref, wj1_ref, bj1_ref, wj2_ref, bj2_ref, out_ref):
    wj2 = wj2_ref[...]
    a0 = _leaky(
        jnp.dot(g0_ref[...], wj1_ref[...], preferred_element_type=jnp.float32) + bj1_ref[...]
    )
    v0 = jnp.sum(a0 * wj2, axis=1, keepdims=True) + bj2_ref[...]
    a1 = _leaky(
        jnp.dot(g1_ref[...], wj1_ref[...], preferred_element_type=jnp.float32) + bj1_ref[...]
    )
    v1 = jnp.sum(a1 * wj2, axis=1, keepdims=True) + bj2_ref[...]
    out_ref[...] = 0.5 * (v0 + v1)


def _tc_jb(gath, wj1, bj1r, wj2r, bj2r):
    blk0 = STEM_P // BS
    blk1 = (STEM_P + JB_P) // BS
    return pl.pallas_call(
        _tc_jb_body,
        grid=(JB_P // BS,),
        in_specs=[
            pl.BlockSpec((BS, DIM), lambda i: (blk0 + i, 0)),
            pl.BlockSpec((BS, DIM), lambda i: (blk1 + i, 0)),
            pl.BlockSpec((DIM, DIM), lambda i: (0, 0)),
            pl.BlockSpec((1, DIM), lambda i: (0, 0)),
            pl.BlockSpec((1, DIM), lambda i: (0, 0)),
            pl.BlockSpec((1, 1), lambda i: (0, 0)),
        ],
        out_specs=pl.BlockSpec((BS, 1), lambda i: (i, 0)),
        out_shape=jax.ShapeDtypeStruct((JB_P, 1), jnp.float32),
    )(gath, gath, wj1, bj1r, wj2r, bj2r)


BN = 2048


def _s2s_q(blih_ref, blhh_ref):
    gates = blih_ref[...] + blhh_ref[...]
    i_ = jax.nn.sigmoid(gates[:, 0:DIM])
    g_ = jnp.tanh(gates[:, 2 * DIM:3 * DIM])
    o_ = jax.nn.sigmoid(gates[:, 3 * DIM:4 * DIM])
    return o_ * jnp.tanh(i_ * g_)


def _s2s_a_body(s_ref, bcol_ref, blih_ref, blhh_ref, emax_ref, acc):
    i = pl.program_id(0)
    q = _s2s_q(blih_ref, blhh_ref)
    e = jnp.sum(s_ref[...] * q, axis=1, keepdims=True)
    gid = lax.broadcasted_iota(jnp.int32, (BN, NG), 1)
    m = bcol_ref[...] == gid
    bm = jnp.max(jnp.where(m, e, -1e30), axis=0, keepdims=True)

    @pl.when(i == 0)
    def _():
        acc[...] = bm

    @pl.when(i > 0)
    def _():
        acc[...] = jnp.maximum(acc[...], bm)

    emax_ref[...] = jnp.where(acc[...] > -1e29, acc[...], 0.0)


def _tc_s2s_a(s10, bcol, blihr, blhhr):
    return pl.pallas_call(
        _s2s_a_body,
        grid=(NP // BN,),
        in_specs=[
            pl.BlockSpec((BN, DIM), lambda i: (i, 0)),
            pl.BlockSpec((BN, 1), lambda i: (i, 0)),
            pl.BlockSpec((1, 4 * DIM), lambda i: (0, 0)),
            pl.BlockSpec((1, 4 * DIM), lambda i: (0, 0)),
        ],
        out_specs=pl.BlockSpec((1, NG), lambda i: (0, 0)),
        out_shape=jax.ShapeDtypeStruct((1, NG), jnp.float32),
        scratch_shapes=[pltpu.VMEM((1, NG), jnp.float32)],
    )(s10, bcol, blihr, blhhr)


def _s2s_b_body(s_ref, bcol_ref, brow_ref, emax_ref, blih_ref, blhh_ref, wo_ref,
                bo_ref, final_ref, asum_acc, rvec_acc):
    i = pl.program_id(0)
    nb = pl.num_programs(0)
    q = _s2s_q(blih_ref, blhh_ref)
    s = s_ref[...]
    e = jnp.sum(s * q, axis=1, keepdims=True)
    gid = lax.broadcasted_iota(jnp.int32, (BN, NG), 1)
    mf = (bcol_ref[...] == gid).astype(jnp.float32)
    emax_pn = jnp.sum(mf * emax_ref[...], axis=1, keepdims=True)
    a = jnp.exp(e - emax_pn)
    gid2 = lax.broadcasted_iota(jnp.int32, (NG, BN), 0)
    mtf = (brow_ref[...] == gid2).astype(jnp.float32)
    asum_blk = jnp.dot(mtf, a, preferred_element_type=jnp.float32)
    rvec_blk = jnp.dot(mtf, a * s, preferred_element_type=jnp.float32)

    @pl.when(i == 0)
    def _():
        asum_acc[...] = jnp.zeros_like(asum_acc)
        rvec_acc[...] = jnp.zeros_like(rvec_acc)

    asum_acc[...] += asum_blk
    rvec_acc[...] += rvec_blk

    @pl.when(i == nb - 1)
    def _():
        asum = asum_acc[...]
        rinv = jnp.where(asum > 0, 1.0 / asum, 0.0)
        rvec = rvec_acc[...] * rinv
        wo = wo_ref[...]
        final_ref[...] = (
            jnp.dot(q, wo[0:DIM, :], preferred_element_type=jnp.float32)
            + jnp.dot(rvec, wo[DIM:2 * DIM, :], preferred_element_type=jnp.float32)
            + bo_ref[...]
        )


def _tc_s2s_b(s10, bcol, brow, emax, blihr, blhhr, wo, bor):
    return pl.pallas_call(
        _s2s_b_body,
        grid=(NP // BN,),
        in_specs=[
            pl.BlockSpec((BN, DIM), lambda i: (i, 0)),
            pl.BlockSpec((BN, 1), lambda i: (i, 0)),
            pl.BlockSpec((1, BN), lambda i: (0, i)),
            pl.BlockSpec((1, NG), lambda i: (0, 0)),
            pl.BlockSpec((1, 4 * DIM), lambda i: (0, 0)),
            pl.BlockSpec((1, 4 * DIM), lambda i: (0, 0)),
            pl.BlockSpec((2 * DIM, 2), lambda i: (0, 0)),
            pl.BlockSpec((1, 2), lambda i: (0, 0)),
        ],
        out_specs=pl.BlockSpec((NG, 2), lambda i: (0, 0)),
        out_shape=jax.ShapeDtypeStruct((NG, 2), jnp.float32),
        scratch_shapes=[
            pltpu.VMEM((NG, 1), jnp.float32),
            pltpu.VMEM((NG, DIM), jnp.float32),
        ],
    )(s10, bcol, brow, emax, blihr, blhhr, wo, bor)


def kernel(x, edge_index, edge_attr, stem_atmidx, jbond_atmidx, batch, W0, b0,
           Wn1, bn1, Wn2, bn2, Wroot, broot, Wih, Whh, bih, bhh, Ws1, bs1, Ws2,
           bs2, Wj1, bj1, Wj2, bj2, Wl_ih, Wl_hh, bl_ih, bl_hh, Wo, bo):
    f32 = jnp.float32
    i32 = jnp.int32
    src = edge_index[0]
    dst = edge_index[1]
    pe = EP - E
    srcp = jnp.concatenate([src, jnp.zeros((pe,), i32)])
    dstp = jnp.concatenate([dst, jnp.full((pe,), NP - 1, i32)])
    eap = jnp.concatenate([edge_attr, jnp.zeros((pe, 4), f32)], axis=0)
    xp = jnp.concatenate([x, jnp.zeros((NP - N, FEAT), f32)], axis=0)
    zpad_s = jnp.zeros((STEM_P - NSTEM,), i32)
    zpad_j = jnp.zeros((JB_P - NJB,), i32)
    ridx = jnp.concatenate([
        stem_atmidx, zpad_s,
        jbond_atmidx[:, 0], zpad_j,
        jbond_atmidx[:, 1], zpad_j,
    ])
    zeros_nd = jnp.zeros((NP, DIM), f32)
    ones_ch = jnp.ones((CH, DIM), f32)
    tf = Wn2.reshape(DIM * DIM, DIM)
    bm = bn2.reshape(DIM, DIM)
    b0r = b0.reshape(1, DIM)
    bn1r = bn1.reshape(1, DIM)
    brootr = broot.reshape(1, DIM)
    bihr = bih.reshape(1, 3 * DIM)
    bhhr = bhh.reshape(1, 3 * DIM)
    bs1r = bs1.reshape(1, DIM)
    bs2r = bs2.reshape(1, NOUT)
    bj1r = bj1.reshape(1, DIM)
    wj2r = Wj2.reshape(1, DIM)
    bj2r = bj2.reshape(1, 1)
    blihr = bl_ih.reshape(1, 4 * DIM)
    blhhr = bl_hh.reshape(1, 4 * DIM)
    bor = bo.reshape(1, 2)
    batchp = jnp.concatenate([batch, jnp.full((NP - N,), -1, i32)])
    bcol = batchp.reshape(NP, 1)
    brow = batchp.reshape(1, NP)

    degp = _sc_deg(dstp, ones_ch, zeros_nd)
    s, dinv = _tc_init(xp, W0, b0r, degp)
    for _ in range(6):
        xs = _sc_gather_edges(s, srcp)
        msg = _tc_msg(eap, xs, Wn1, bn1r, tf, bm)
        aggp = _sc_scatter(msg, dstp, zeros_nd)
        s = _tc_gru(s, aggp, dinv, Wroot, brootr, Wih, bihr, Whh, bhhr)
    gath = _sc_gather_read(s, ridx)
    stem = _tc_stem(gath, Ws1, bs1r, Ws2, bs2r)
    jb = _tc_jb(gath, Wj1, bj1r, wj2r, bj2r)
    emax = _tc_s2s_a(s, bcol, blihr, blhhr)
    final = _tc_s2s_b(s, bcol, brow, emax, blihr, blhhr, Wo, bor)
    return final, stem[:NSTEM], jb[:NJB, 0]


# speedup vs baseline: 2.5112x; 2.5112x over previous
import functools

import jax
import jax.numpy as jnp
from jax import lax
from jax.experimental import pallas as pl
from jax.experimental.pallas import tpu as pltpu
from jax.experimental.pallas import tpu_sc as plsc

N = 10000
E = 160000
FEAT = 14
DIM = 32
NOUT = 105
NG = 256
NSTEM = 20000
NJB = 10000

NP = 10240
EP = 163840
NC = 2
NS = 16
NW = NC * NS
CH = 1024
ECHUNKS = EP // NW // CH
NSLC = NP // NS

STEM_P = 20480
JB_P = 10240
RLEN = STEM_P + 2 * JB_P
RCH = 1280
RCHUNKS = RLEN // NW // RCH

_mesh = plsc.VectorSubcoreMesh(core_axis_name="c", subcore_axis_name="s")
_sc_params = pltpu.CompilerParams(use_tc_tiling_on_sc=False)


def _leaky(v):
    return jnp.where(v >= 0, v, 0.01 * v)


def _sc_gather_body(nchunks, chunk, per_worker, table, idx_hbm, out_hbm, idxv, rowsv, sem):
    cid = lax.axis_index("c")
    sid = lax.axis_index("s")
    wid = sid * NC + cid
    for ci in range(nchunks):
        off = wid * per_worker + ci * chunk
        pltpu.sync_copy(idx_hbm.at[pl.ds(off, chunk)], idxv)
        pltpu.async_copy(table.at[idxv], rowsv, sem).wait()
        pltpu.sync_copy(rowsv, out_hbm.at[pl.ds(off, chunk)])


def _make_sc_gather(total, nchunks, chunk):
    per_worker = total // NW
    return functools.partial(
        pl.kernel,
        out_type=jax.ShapeDtypeStruct((total, DIM), jnp.float32),
        mesh=_mesh,
        scratch_types=[
            pltpu.VMEM((chunk,), jnp.int32),
            pltpu.VMEM((chunk, DIM), jnp.float32),
            pltpu.SemaphoreType.DMA,
        ],
        compiler_params=_sc_params,
    )(functools.partial(_sc_gather_body, nchunks, chunk, per_worker))


_sc_gather_edges = _make_sc_gather(EP, ECHUNKS, CH)
_sc_gather_read = _make_sc_gather(RLEN, RCHUNKS, RCH)


def _sc_scatter_body(msg_hbm, dst_hbm, zeros_hbm, aggp_hbm, idxv, msgv, acc):
    cid = lax.axis_index("c")
    sid = lax.axis_index("s")
    pltpu.sync_copy(zeros_hbm.at[pl.ds(sid * NSLC, NSLC)], acc.at[pl.ds(sid * NSLC, NSLC)])
    plsc.subcore_barrier()
    half = EP // NC
    for ci in range(ECHUNKS):
        off = cid * half + sid * (half // NS) + ci * CH
        pltpu.sync_copy(dst_hbm.at[pl.ds(off, CH)], idxv)
        pltpu.sync_copy(msg_hbm.at[pl.ds(off, CH)], msgv)
        pltpu.sync_copy(msgv, acc.at[idxv], add=True)
    plsc.subcore_barrier()
    pltpu.sync_copy(acc.at[pl.ds(sid * NSLC, NSLC)], aggp_hbm.at[cid, pl.ds(sid * NSLC, NSLC)])


_sc_scatter = functools.partial(
    pl.kernel,
    out_type=jax.ShapeDtypeStruct((NC, NP, DIM), jnp.float32),
    mesh=_mesh,
    scratch_types=[
        pltpu.VMEM((CH,), jnp.int32),
        pltpu.VMEM((CH, DIM), jnp.float32),
        pltpu.VMEM_SHARED((NP, DIM), jnp.float32),
    ],
    compiler_params=_sc_params,
)(_sc_scatter_body)


def _sc_deg_body(dst_hbm, ones_hbm, zeros_hbm, degp_hbm, idxv, onesv, acc):
    cid = lax.axis_index("c")
    sid = lax.axis_index("s")
    pltpu.sync_copy(zeros_hbm.at[pl.ds(sid * NSLC, NSLC)], acc.at[pl.ds(sid * NSLC, NSLC)])
    pltpu.sync_copy(ones_hbm, onesv)
    plsc.subcore_barrier()
    half = EP // NC
    for ci in range(ECHUNKS):
        off = cid * half + sid * (half // NS) + ci * CH
        pltpu.sync_copy(dst_hbm.at[pl.ds(off, CH)], idxv)
        pltpu.sync_copy(onesv, acc.at[idxv], add=True)
    plsc.subcore_barrier()
    pltpu.sync_copy(acc.at[pl.ds(sid * NSLC, NSLC)], degp_hbm.at[cid, pl.ds(sid * NSLC, NSLC)])


_sc_deg = functools.partial(
    pl.kernel,
    out_type=jax.ShapeDtypeStruct((NC, NP, DIM), jnp.float32),
    mesh=_mesh,
    scratch_types=[
        pltpu.VMEM((CH,), jnp.int32),
        pltpu.VMEM((CH, DIM), jnp.float32),
        pltpu.VMEM_SHARED((NP, DIM), jnp.float32),
    ],
    compiler_params=_sc_params,
)(_sc_deg_body)


def _tc_init_body(xp_ref, w0_ref, b0_ref, degp_ref, s0_ref, dinv_ref):
    s0_ref[...] = _leaky(
        jnp.dot(xp_ref[...], w0_ref[...], preferred_element_type=jnp.float32) + b0_ref[...]
    )
    deg = jnp.maximum(degp_ref[0] + degp_ref[1], 1.0)
    dinv_ref[...] = 1.0 / deg


def _tc_init(xp, w0, b0r, degp):
    return pl.pallas_call(
        _tc_init_body,
        out_shape=(
            jax.ShapeDtypeStruct((NP, DIM), jnp.float32),
            jax.ShapeDtypeStruct((NP, DIM), jnp.float32),
        ),
    )(xp, w0, b0r, degp)


BE = 1024


def _split_hi_lo(v):
    hi = v.astype(jnp.bfloat16).astype(jnp.float32)
    return hi, v - hi


def _tc_msg_body(ea_ref, xs_ref, wn13_ref, bn1_ref, tcat3_ref, sel_ref, msg_ref):
    ea = ea_ref[...]
    ea_hi, ea_lo = _split_hi_lo(ea)
    ea3 = jnp.concatenate([ea_hi, ea_hi, ea_lo], axis=1)
    z = _leaky(
        jnp.dot(ea3, wn13_ref[...], preferred_element_type=jnp.float32) + bn1_ref[...]
    )
    xs = xs_ref[...]
    xs_hi, xs_lo = _split_hi_lo(xs)
    xs3 = jnp.concatenate([xs_hi, xs_hi, xs_lo], axis=1)
    h = jnp.dot(xs3, tcat3_ref[...], preferred_element_type=jnp.float32)
    p = h[:, :DIM * DIM] * jnp.tile(z, (1, DIM))
    msg_ref[...] = (
        jnp.dot(p, sel_ref[...], preferred_element_type=jnp.float32)
        + h[:, DIM * DIM:]
    )


def _tc_msg(eap, xs, wn13, bn1r, tcat3, sel):
    nblk = EP // BE
    return pl.pallas_call(
        _tc_msg_body,
        grid=(nblk,),
        in_specs=[
            pl.BlockSpec((BE, 4), lambda i: (i, 0)),
            pl.BlockSpec((BE, DIM), lambda i: (i, 0)),
            pl.BlockSpec((12, DIM), lambda i: (0, 0)),
            pl.BlockSpec((1, DIM), lambda i: (0, 0)),
            pl.BlockSpec((3 * DIM, DIM * DIM + DIM), lambda i: (0, 0)),
            pl.BlockSpec((DIM * DIM, DIM), lambda i: (0, 0)),
        ],
        out_specs=pl.BlockSpec((BE, DIM), lambda i: (i, 0)),
        out_shape=jax.ShapeDtypeStruct((EP, DIM), jnp.float32),
    )(eap, xs, wn13, bn1r, tcat3, sel)


def _tc_gru_body(s_ref, aggp_ref, dinv_ref, wroot_ref, broot_ref, wih_ref, bih_ref,
                 whh_ref, bhh_ref, out_ref):
    s = s_ref[...]
    agg = (aggp_ref[0] + aggp_ref[1]) * dinv_ref[...]
    m = _leaky(
        jnp.dot(s, wroot_ref[...], preferred_element_type=jnp.float32)
        + broot_ref[...] + agg
    )
    gi = jnp.dot(m, wih_ref[...], preferred_element_type=jnp.float32) + bih_ref[...]
    gh = jnp.dot(s, whh_ref[...], preferred_element_type=jnp.float32) + bhh_ref[...]
    r = jax.nn.sigmoid(gi[:, 0:DIM] + gh[:, 0:DIM])
    zz = jax.nn.sigmoid(gi[:, DIM:2 * DIM] + gh[:, DIM:2 * DIM])
    n = jnp.tanh(gi[:, 2 * DIM:3 * DIM] + r * gh[:, 2 * DIM:3 * DIM])
    out_ref[...] = (1.0 - zz) * n + zz * s


def _tc_gru(s, aggp, dinv, wroot, brootr, wih, bihr, whh, bhhr):
    return pl.pallas_call(
        _tc_gru_body,
        out_shape=jax.ShapeDtypeStruct((NP, DIM), jnp.float32),
    )(s, aggp, dinv, wroot, brootr, wih, bihr, whh, bhhr)


BS = 2048


def _tc_stem_body(g_ref, ws1_ref, bs1_ref, ws2_ref, bs2_ref, out_ref):
    act = _leaky(
        jnp.dot(g_ref[...], ws1_ref[...], preferred_element_type=jnp.float32) + bs1_ref[...]
    )
    out_ref[...] = jnp.dot(act, ws2_ref[...], preferred_element_type=jnp.float32) + bs2_ref[...]


def _tc_stem(gath, ws1, bs1r, ws2, bs2r):
    return pl.pallas_call(
        _tc_stem_body,
        grid=(STEM_P // BS,),
        in_specs=[
            pl.BlockSpec((BS, DIM), lambda i: (i, 0)),
            pl.BlockSpec((DIM, DIM), lambda i: (0, 0)),
            pl.BlockSpec((1, DIM), lambda i: (0, 0)),
            pl.BlockSpec((DIM, NOUT), lambda i: (0, 0)),
            pl.BlockSpec((1, NOUT), lambda i: (0, 0)),
        ],
        out_specs=pl.BlockSpec((BS, NOUT), lambda i: (i, 0)),
        out_shape=jax.ShapeDtypeStruct((STEM_P, NOUT), jnp.float32),
    )(gath, ws1, bs1r, ws2, bs2r)


def _tc_jb_body(g0_ref, g1_ref, wj1_ref, bj1_ref, wj2_ref, bj2_ref, out_ref):
    wj2 = wj2_ref[...]
    a0 = _leaky(
        jnp.dot(g0_ref[...], wj1_ref[...], preferred_element_type=jnp.float32) + bj1_ref[...]
    )
    v0 = jnp.sum(a0 * wj2, axis=1, keepdims=True) + bj2_ref[...]
    a1 = _leaky(
        jnp.dot(g1_ref[...], wj1_ref[...], preferred_element_type=jnp.float32) + bj1_ref[...]
    )
    v1 = jnp.sum(a1 * wj2, axis=1, keepdims=True) + bj2_ref[...]
    out_ref[...] = 0.5 * (v0 + v1)


def _tc_jb(gath, wj1, bj1r, wj2r, bj2r):
    blk0 = STEM_P // BS
    blk1 = (STEM_P + JB_P) // BS
    return pl.pallas_call(
        _tc_jb_body,
        grid=(JB_P // BS,),
        in_specs=[
            pl.BlockSpec((BS, DIM), lambda i: (blk0 + i, 0)),
            pl.BlockSpec((BS, DIM), lambda i: (blk1 + i, 0)),
            pl.BlockSpec((DIM, DIM), lambda i: (0, 0)),
            pl.BlockSpec((1, DIM), lambda i: (0, 0)),
            pl.BlockSpec((1, DIM), lambda i: (0, 0)),
            pl.BlockSpec((1, 1), lambda i: (0, 0)),
        ],
        out_specs=pl.BlockSpec((BS, 1), lambda i: (i, 0)),
        out_shape=jax.ShapeDtypeStruct((JB_P, 1), jnp.float32),
    )(gath, gath, wj1, bj1r, wj2r, bj2r)


BN = 2048


def _s2s_q(blih_ref, blhh_ref):
    gates = blih_ref[...] + blhh_ref[...]
    i_ = jax.nn.sigmoid(gates[:, 0:DIM])
    g_ = jnp.tanh(gates[:, 2 * DIM:3 * DIM])
    o_ = jax.nn.sigmoid(gates[:, 3 * DIM:4 * DIM])
    return o_ * jnp.tanh(i_ * g_)


def _s2s_a_body(s_ref, bcol_ref, blih_ref, blhh_ref, emax_ref, acc):
    i = pl.program_id(0)
    q = _s2s_q(blih_ref, blhh_ref)
    e = jnp.sum(s_ref[...] * q, axis=1, keepdims=True)
    gid = lax.broadcasted_iota(jnp.int32, (BN, NG), 1)
    m = bcol_ref[...] == gid
    bm = jnp.max(jnp.where(m, e, -1e30), axis=0, keepdims=True)

    @pl.when(i == 0)
    def _():
        acc[...] = bm

    @pl.when(i > 0)
    def _():
        acc[...] = jnp.maximum(acc[...], bm)

    emax_ref[...] = jnp.where(acc[...] > -1e29, acc[...], 0.0)


def _tc_s2s_a(s10, bcol, blihr, blhhr):
    return pl.pallas_call(
        _s2s_a_body,
        grid=(NP // BN,),
        in_specs=[
            pl.BlockSpec((BN, DIM), lambda i: (i, 0)),
            pl.BlockSpec((BN, 1), lambda i: (i, 0)),
            pl.BlockSpec((1, 4 * DIM), lambda i: (0, 0)),
            pl.BlockSpec((1, 4 * DIM), lambda i: (0, 0)),
        ],
        out_specs=pl.BlockSpec((1, NG), lambda i: (0, 0)),
        out_shape=jax.ShapeDtypeStruct((1, NG), jnp.float32),
        scratch_shapes=[pltpu.VMEM((1, NG), jnp.float32)],
    )(s10, bcol, blihr, blhhr)


def _s2s_b_body(s_ref, bcol_ref, brow_ref, emax_ref, blih_ref, blhh_ref, wo_ref,
                bo_ref, final_ref, asum_acc, rvec_acc):
    i = pl.program_id(0)
    nb = pl.num_programs(0)
    q = _s2s_q(blih_ref, blhh_ref)
    s = s_ref[...]
    e = jnp.sum(s * q, axis=1, keepdims=True)
    gid = lax.broadcasted_iota(jnp.int32, (BN, NG), 1)
    mf = (bcol_ref[...] == gid).astype(jnp.float32)
    emax_pn = jnp.sum(mf * emax_ref[...], axis=1, keepdims=True)
    a = jnp.exp(e - emax_pn)
    gid2 = lax.broadcasted_iota(jnp.int32, (NG, BN), 0)
    mtf = (brow_ref[...] == gid2).astype(jnp.float32)
    asum_blk = jnp.dot(mtf, a, preferred_element_type=jnp.float32)
    rvec_blk = jnp.dot(mtf, a * s, preferred_element_type=jnp.float32)

    @pl.when(i == 0)
    def _():
        asum_acc[...] = jnp.zeros_like(asum_acc)
        rvec_acc[...] = jnp.zeros_like(rvec_acc)

    asum_acc[...] += asum_blk
    rvec_acc[...] += rvec_blk

    @pl.when(i == nb - 1)
    def _():
        asum = asum_acc[...]
        rinv = jnp.where(asum > 0, 1.0 / asum, 0.0)
        rvec = rvec_acc[...] * rinv
        wo = wo_ref[...]
        final_ref[...] = (
            jnp.dot(q, wo[0:DIM, :], preferred_element_type=jnp.float32)
            + jnp.dot(rvec, wo[DIM:2 * DIM, :], preferred_element_type=jnp.float32)
            + bo_ref[...]
        )


def _tc_s2s_b(s10, bcol, brow, emax, blihr, blhhr, wo, bor):
    return pl.pallas_call(
        _s2s_b_body,
        grid=(NP // BN,),
        in_specs=[
            pl.BlockSpec((BN, DIM), lambda i: (i, 0)),
            pl.BlockSpec((BN, 1), lambda i: (i, 0)),
            pl.BlockSpec((1, BN), lambda i: (0, i)),
            pl.BlockSpec((1, NG), lambda i: (0, 0)),
            pl.BlockSpec((1, 4 * DIM), lambda i: (0, 0)),
            pl.BlockSpec((1, 4 * DIM), lambda i: (0, 0)),
            pl.BlockSpec((2 * DIM, 2), lambda i: (0, 0)),
            pl.BlockSpec((1, 2), lambda i: (0, 0)),
        ],
        out_specs=pl.BlockSpec((NG, 2), lambda i: (0, 0)),
        out_shape=jax.ShapeDtypeStruct((NG, 2), jnp.float32),
        scratch_shapes=[
            pltpu.VMEM((NG, 1), jnp.float32),
            pltpu.VMEM((NG, DIM), jnp.float32),
        ],
    )(s10, bcol, brow, emax, blihr, blhhr, wo, bor)


def kernel(x, edge_index, edge_attr, stem_atmidx, jbond_atmidx, batch, W0, b0,
           Wn1, bn1, Wn2, bn2, Wroot, broot, Wih, Whh, bih, bhh, Ws1, bs1, Ws2,
           bs2, Wj1, bj1, Wj2, bj2, Wl_ih, Wl_hh, bl_ih, bl_hh, Wo, bo):
    f32 = jnp.float32
    i32 = jnp.int32
    src = edge_index[0]
    dst = edge_index[1]
    pe = EP - E
    srcp = jnp.concatenate([src, jnp.zeros((pe,), i32)])
    dstp = jnp.concatenate([dst, jnp.full((pe,), NP - 1, i32)])
    eap = jnp.concatenate([edge_attr, jnp.zeros((pe, 4), f32)], axis=0)
    xp = jnp.concatenate([x, jnp.zeros((NP - N, FEAT), f32)], axis=0)
    zpad_s = jnp.zeros((STEM_P - NSTEM,), i32)
    zpad_j = jnp.zeros((JB_P - NJB,), i32)
    ridx = jnp.concatenate([
        stem_atmidx, zpad_s,
        jbond_atmidx[:, 0], zpad_j,
        jbond_atmidx[:, 1], zpad_j,
    ])
    zeros_nd = jnp.zeros((NP, DIM), f32)
    ones_ch = jnp.ones((CH, DIM), f32)

    def _hl(v):
        hi = v.astype(jnp.bfloat16).astype(f32)
        return hi, v - hi

    tcat = jnp.transpose(Wn2.reshape(DIM, DIM, DIM), (1, 2, 0)).reshape(DIM, DIM * DIM)
    tcatb = jnp.concatenate([tcat, bn2.reshape(DIM, DIM)], axis=1)
    tb_hi, tb_lo = _hl(tcatb)
    tcat3 = jnp.concatenate([tb_hi, tb_lo, tb_hi], axis=0)
    w1_hi, w1_lo = _hl(Wn1)
    wn13 = jnp.concatenate([w1_hi, w1_lo, w1_hi], axis=0)
    sel = jnp.repeat(jnp.eye(DIM, dtype=f32), DIM, axis=0)
    b0r = b0.reshape(1, DIM)
    bn1r = bn1.reshape(1, DIM)
    brootr = broot.reshape(1, DIM)
    bihr = bih.reshape(1, 3 * DIM)
    bhhr = bhh.reshape(1, 3 * DIM)
    bs1r = bs1.reshape(1, DIM)
    bs2r = bs2.reshape(1, NOUT)
    bj1r = bj1.reshape(1, DIM)
    wj2r = Wj2.reshape(1, DIM)
    bj2r = bj2.reshape(1, 1)
    blihr = bl_ih.reshape(1, 4 * DIM)
    blhhr = bl_hh.reshape(1, 4 * DIM)
    bor = bo.reshape(1, 2)
    batchp = jnp.concatenate([batch, jnp.full((NP - N,), -1, i32)])
    bcol = batchp.reshape(NP, 1)
    brow = batchp.reshape(1, NP)

    degp = _sc_deg(dstp, ones_ch, zeros_nd)
    s, dinv = _tc_init(xp, W0, b0r, degp)
    for _ in range(6):
        xs = _sc_gather_edges(s, srcp)
        msg = _tc_msg(eap, xs, wn13, bn1r, tcat3, sel)
        aggp = _sc_scatter(msg, dstp, zeros_nd)
        s = _tc_gru(s, aggp, dinv, Wroot, brootr, Wih, bihr, Whh, bhhr)
    gath = _sc_gather_read(s, ridx)
    stem = _tc_stem(gath, Ws1, bs1r, Ws2, bs2r)
    jb = _tc_jb(gath, Wj1, bj1r, wj2r, bj2r)
    emax = _tc_s2s_a(s, bcol, blihr, blhhr)
    final = _tc_s2s_b(s, bcol, brow, emax, blihr, blhhr, Wo, bor)
    return final, stem[:NSTEM], jb[:NJB, 0]


# speedup vs baseline: 2.5590x; 1.0191x over previous
import functools

import jax
import jax.numpy as jnp
from jax import lax
from jax.experimental import pallas as pl
from jax.experimental.pallas import tpu as pltpu
from jax.experimental.pallas import tpu_sc as plsc

N = 10000
E = 160000
FEAT = 14
DIM = 32
NOUT = 105
NG = 256
NSTEM = 20000
NJB = 10000

NP = 10240
EP = 163840
NC = 2
NS = 16
NW = NC * NS
CH = 1024
ECHUNKS = EP // NW // CH
NSLC = NP // NS

STEM_P = 20480
JB_P = 10240
RLEN = STEM_P + 2 * JB_P
RCH = 1280
RCHUNKS = RLEN // NW // RCH

_mesh = plsc.VectorSubcoreMesh(core_axis_name="c", subcore_axis_name="s")
_sc_params = pltpu.CompilerParams(use_tc_tiling_on_sc=False)


def _leaky(v):
    return jnp.where(v >= 0, v, 0.01 * v)


def _sc_gather_body(nchunks, chunk, per_worker, table, idx_hbm, out_hbm,
                    i0, i1, r0, r1, sg0, sg1, sw0, sw1):
    cid = lax.axis_index("c")
    sid = lax.axis_index("s")
    wid = sid * NC + cid
    base = wid * per_worker
    ib, rb = [i0, i1], [r0, r1]
    sg, sw = [sg0, sg1], [sw0, sw1]
    g, w = {}, {}
    pltpu.sync_copy(idx_hbm.at[pl.ds(base, chunk)], i0)
    g[0] = pltpu.async_copy(table.at[i0], r0, sg0)
    for c in range(nchunks):
        cur, nxt = c % 2, (c + 1) % 2
        if c + 1 < nchunks:
            pltpu.sync_copy(idx_hbm.at[pl.ds(base + (c + 1) * chunk, chunk)], ib[nxt])
            if c >= 1:
                w[c - 1].wait()
            g[c + 1] = pltpu.async_copy(table.at[ib[nxt]], rb[nxt], sg[nxt])
        elif c >= 1:
            w[c - 1].wait()
        g[c].wait()
        w[c] = pltpu.async_copy(rb[cur], out_hbm.at[pl.ds(base + c * chunk, chunk)], sw[cur])
    w[nchunks - 1].wait()


def _make_sc_gather(total, nchunks, chunk):
    per_worker = total // NW
    return functools.partial(
        pl.kernel,
        out_type=jax.ShapeDtypeStruct((total, DIM), jnp.float32),
        mesh=_mesh,
        scratch_types=[
            pltpu.VMEM((chunk,), jnp.int32),
            pltpu.VMEM((chunk,), jnp.int32),
            pltpu.VMEM((chunk, DIM), jnp.float32),
            pltpu.VMEM((chunk, DIM), jnp.float32),
            pltpu.SemaphoreType.DMA,
            pltpu.SemaphoreType.DMA,
            pltpu.SemaphoreType.DMA,
            pltpu.SemaphoreType.DMA,
        ],
        compiler_params=_sc_params,
    )(functools.partial(_sc_gather_body, nchunks, chunk, per_worker))


_sc_gather_edges = _make_sc_gather(EP, ECHUNKS, CH)
_sc_gather_read = _make_sc_gather(RLEN, RCHUNKS, RCH)


def _sc_scatter_body(msg_hbm, dst_hbm, zeros_hbm, aggp_hbm, d0, d1, m0, m1, acc, sm0, sm1):
    cid = lax.axis_index("c")
    sid = lax.axis_index("s")
    pltpu.sync_copy(zeros_hbm.at[pl.ds(sid * NSLC, NSLC)], acc.at[pl.ds(sid * NSLC, NSLC)])
    half = EP // NC
    base = cid * half + sid * (half // NS)
    db, mb, sm = [d0, d1], [m0, m1], [sm0, sm1]
    a = {}
    pltpu.sync_copy(dst_hbm.at[pl.ds(base, CH)], d0)
    a[0] = pltpu.async_copy(msg_hbm.at[pl.ds(base, CH)], m0, sm0)
    plsc.subcore_barrier()
    for c in range(ECHUNKS):
        cur, nxt = c % 2, (c + 1) % 2
        if c + 1 < ECHUNKS:
            pltpu.sync_copy(dst_hbm.at[pl.ds(base + (c + 1) * CH, CH)], db[nxt])
            a[c + 1] = pltpu.async_copy(msg_hbm.at[pl.ds(base + (c + 1) * CH, CH)], mb[nxt], sm[nxt])
        a[c].wait()
        pltpu.sync_copy(mb[cur], acc.at[db[cur]], add=True)
    plsc.subcore_barrier()
    pltpu.sync_copy(acc.at[pl.ds(sid * NSLC, NSLC)], aggp_hbm.at[cid, pl.ds(sid * NSLC, NSLC)])


_sc_scatter = functools.partial(
    pl.kernel,
    out_type=jax.ShapeDtypeStruct((NC, NP, DIM), jnp.float32),
    mesh=_mesh,
    scratch_types=[
        pltpu.VMEM((CH,), jnp.int32),
        pltpu.VMEM((CH,), jnp.int32),
        pltpu.VMEM((CH, DIM), jnp.float32),
        pltpu.VMEM((CH, DIM), jnp.float32),
        pltpu.VMEM_SHARED((NP, DIM), jnp.float32),
        pltpu.SemaphoreType.DMA,
        pltpu.SemaphoreType.DMA,
    ],
    compiler_params=_sc_params,
)(_sc_scatter_body)


def _sc_deg_body(dst_hbm, ones_hbm, zeros_hbm, degp_hbm, idxv, onesv, acc):
    cid = lax.axis_index("c")
    sid = lax.axis_index("s")
    pltpu.sync_copy(zeros_hbm.at[pl.ds(sid * NSLC, NSLC)], acc.at[pl.ds(sid * NSLC, NSLC)])
    pltpu.sync_copy(ones_hbm, onesv)
    plsc.subcore_barrier()
    half = EP // NC
    for ci in range(ECHUNKS):
        off = cid * half + sid * (half // NS) + ci * CH
        pltpu.sync_copy(dst_hbm.at[pl.ds(off, CH)], idxv)
        pltpu.sync_copy(onesv, acc.at[idxv], add=True)
    plsc.subcore_barrier()
    pltpu.sync_copy(acc.at[pl.ds(sid * NSLC, NSLC)], degp_hbm.at[cid, pl.ds(sid * NSLC, NSLC)])


_sc_deg = functools.partial(
    pl.kernel,
    out_type=jax.ShapeDtypeStruct((NC, NP, DIM), jnp.float32),
    mesh=_mesh,
    scratch_types=[
        pltpu.VMEM((CH,), jnp.int32),
        pltpu.VMEM((CH, DIM), jnp.float32),
        pltpu.VMEM_SHARED((NP, DIM), jnp.float32),
    ],
    compiler_params=_sc_params,
)(_sc_deg_body)


def _tc_init_body(xp_ref, w0_ref, b0_ref, degp_ref, s0_ref, dinv_ref):
    s0_ref[...] = _leaky(
        jnp.dot(xp_ref[...], w0_ref[...], preferred_element_type=jnp.float32) + b0_ref[...]
    )
    deg = jnp.maximum(degp_ref[0] + degp_ref[1], 1.0)
    dinv_ref[...] = 1.0 / deg


def _tc_init(xp, w0, b0r, degp):
    return pl.pallas_call(
        _tc_init_body,
        out_shape=(
            jax.ShapeDtypeStruct((NP, DIM), jnp.float32),
            jax.ShapeDtypeStruct((NP, DIM), jnp.float32),
        ),
    )(xp, w0, b0r, degp)


BE = 1024


def _split_hi_lo(v):
    hi = v.astype(jnp.bfloat16).astype(jnp.float32)
    return hi, v - hi


def _tc_msg_body(ea_ref, xs_ref, wn13_ref, bn1_ref, tcat3_ref, sel_ref, msg_ref):
    ea = ea_ref[...]
    ea_hi, ea_lo = _split_hi_lo(ea)
    ea3 = jnp.concatenate([ea_hi, ea_hi, ea_lo], axis=1)
    z = _leaky(
        jnp.dot(ea3, wn13_ref[...], preferred_element_type=jnp.float32) + bn1_ref[...]
    )
    xs = xs_ref[...]
    xs_hi, xs_lo = _split_hi_lo(xs)
    xs3 = jnp.concatenate([xs_hi, xs_hi, xs_lo], axis=1)
    h = jnp.dot(xs3, tcat3_ref[...], preferred_element_type=jnp.float32)
    p = h[:, :DIM * DIM] * jnp.tile(z, (1, DIM))
    msg_ref[...] = (
        jnp.dot(p, sel_ref[...], preferred_element_type=jnp.float32)
        + h[:, DIM * DIM:]
    )


def _tc_msg(eap, xs, wn13, bn1r, tcat3, sel):
    nblk = EP // BE
    return pl.pallas_call(
        _tc_msg_body,
        grid=(nblk,),
        in_specs=[
            pl.BlockSpec((BE, 4), lambda i: (i, 0)),
            pl.BlockSpec((BE, DIM), lambda i: (i, 0)),
            pl.BlockSpec((12, DIM), lambda i: (0, 0)),
            pl.BlockSpec((1, DIM), lambda i: (0, 0)),
            pl.BlockSpec((3 * DIM, DIM * DIM + DIM), lambda i: (0, 0)),
            pl.BlockSpec((DIM * DIM, DIM), lambda i: (0, 0)),
        ],
        out_specs=pl.BlockSpec((BE, DIM), lambda i: (i, 0)),
        out_shape=jax.ShapeDtypeStruct((EP, DIM), jnp.float32),
    )(eap, xs, wn13, bn1r, tcat3, sel)


def _tc_gru_body(s_ref, aggp_ref, dinv_ref, wroot_ref, broot_ref, wih_ref, bih_ref,
                 whh_ref, bhh_ref, out_ref):
    s = s_ref[...]
    agg = (aggp_ref[0] + aggp_ref[1]) * dinv_ref[...]
    m = _leaky(
        jnp.dot(s, wroot_ref[...], preferred_element_type=jnp.float32)
        + broot_ref[...] + agg
    )
    gi = jnp.dot(m, wih_ref[...], preferred_element_type=jnp.float32) + bih_ref[...]
    gh = jnp.dot(s, whh_ref[...], preferred_element_type=jnp.float32) + bhh_ref[...]
    r = jax.nn.sigmoid(gi[:, 0:DIM] + gh[:, 0:DIM])
    zz = jax.nn.sigmoid(gi[:, DIM:2 * DIM] + gh[:, DIM:2 * DIM])
    n = jnp.tanh(gi[:, 2 * DIM:3 * DIM] + r * gh[:, 2 * DIM:3 * DIM])
    out_ref[...] = (1.0 - zz) * n + zz * s


def _tc_gru(s, aggp, dinv, wroot, brootr, wih, bihr, whh, bhhr):
    return pl.pallas_call(
        _tc_gru_body,
        out_shape=jax.ShapeDtypeStruct((NP, DIM), jnp.float32),
    )(s, aggp, dinv, wroot, brootr, wih, bihr, whh, bhhr)


BS = 2048


def _tc_stem_body(g_ref, ws1_ref, bs1_ref, ws2_ref, bs2_ref, out_ref):
    act = _leaky(
        jnp.dot(g_ref[...], ws1_ref[...], preferred_element_type=jnp.float32) + bs1_ref[...]
    )
    out_ref[...] = jnp.dot(act, ws2_ref[...], preferred_element_type=jnp.float32) + bs2_ref[...]


def _tc_stem(gath, ws1, bs1r, ws2, bs2r):
    return pl.pallas_call(
        _tc_stem_body,
        grid=(STEM_P // BS,),
        in_specs=[
            pl.BlockSpec((BS, DIM), lambda i: (i, 0)),
            pl.BlockSpec((DIM, DIM), lambda i: (0, 0)),
            pl.BlockSpec((1, DIM), lambda i: (0, 0)),
            pl.BlockSpec((DIM, NOUT), lambda i: (0, 0)),
            pl.BlockSpec((1, NOUT), lambda i: (0, 0)),
        ],
        out_specs=pl.BlockSpec((BS, NOUT), lambda i: (i, 0)),
        out_shape=jax.ShapeDtypeStruct((STEM_P, NOUT), jnp.float32),
    )(gath, ws1, bs1r, ws2, bs2r)


def _tc_jb_body(g0_ref, g1_ref, wj1_ref, bj1_ref, wj2_ref, bj2_ref, out_ref):
    wj2 = wj2_ref[...]
    a0 = _leaky(
        jnp.dot(g0_ref[...], wj1_ref[...], preferred_element_type=jnp.float32) + bj1_ref[...]
    )
    v0 = jnp.sum(a0 * wj2, axis=1, keepdims=True) + bj2_ref[...]
    a1 = _leaky(
        jnp.dot(g1_ref[...], wj1_ref[...], preferred_element_type=jnp.float32) + bj1_ref[...]
    )
    v1 = jnp.sum(a1 * wj2, axis=1, keepdims=True) + bj2_ref[...]
    out_ref[...] = 0.5 * (v0 + v1)


def _tc_jb(gath, wj1, bj1r, wj2r, bj2r):
    blk0 = STEM_P // BS
    blk1 = (STEM_P + JB_P) // BS
    return pl.pallas_call(
        _tc_jb_body,
        grid=(JB_P // BS,),
        in_specs=[
            pl.BlockSpec((BS, DIM), lambda i: (blk0 + i, 0)),
            pl.BlockSpec((BS, DIM), lambda i: (blk1 + i, 0)),
            pl.BlockSpec((DIM, DIM), lambda i: (0, 0)),
            pl.BlockSpec((1, DIM), lambda i: (0, 0)),
            pl.BlockSpec((1, DIM), lambda i: (0, 0)),
            pl.BlockSpec((1, 1), lambda i: (0, 0)),
        ],
        out_specs=pl.BlockSpec((BS, 1), lambda i: (i, 0)),
        out_shape=jax.ShapeDtypeStruct((JB_P, 1), jnp.float32),
    )(gath, gath, wj1, bj1r, wj2r, bj2r)


BN = 2048


def _s2s_q(blih_ref, blhh_ref):
    gates = blih_ref[...] + blhh_ref[...]
    i_ = jax.nn.sigmoid(gates[:, 0:DIM])
    g_ = jnp.tanh(gates[:, 2 * DIM:3 * DIM])
    o_ = jax.nn.sigmoid(gates[:, 3 * DIM:4 * DIM])
    return o_ * jnp.tanh(i_ * g_)


def _s2s_a_body(s_ref, bcol_ref, blih_ref, blhh_ref, emax_ref, acc):
    i = pl.program_id(0)
    q = _s2s_q(blih_ref, blhh_ref)
    e = jnp.sum(s_ref[...] * q, axis=1, keepdims=True)
    gid = lax.broadcasted_iota(jnp.int32, (BN, NG), 1)
    m = bcol_ref[...] == gid
    bm = jnp.max(jnp.where(m, e, -1e30), axis=0, keepdims=True)

    @pl.when(i == 0)
    def _():
        acc[...] = bm

    @pl.when(i > 0)
    def _():
        acc[...] = jnp.maximum(acc[...], bm)

    emax_ref[...] = jnp.where(acc[...] > -1e29, acc[...], 0.0)


def _tc_s2s_a(s10, bcol, blihr, blhhr):
    return pl.pallas_call(
        _s2s_a_body,
        grid=(NP // BN,),
        in_specs=[
            pl.BlockSpec((BN, DIM), lambda i: (i, 0)),
            pl.BlockSpec((BN, 1), lambda i: (i, 0)),
            pl.BlockSpec((1, 4 * DIM), lambda i: (0, 0)),
            pl.BlockSpec((1, 4 * DIM), lambda i: (0, 0)),
        ],
        out_specs=pl.BlockSpec((1, NG), lambda i: (0, 0)),
        out_shape=jax.ShapeDtypeStruct((1, NG), jnp.float32),
        scratch_shapes=[pltpu.VMEM((1, NG), jnp.float32)],
    )(s10, bcol, blihr, blhhr)


def _s2s_b_body(s_ref, bcol_ref, brow_ref, emax_ref, blih_ref, blhh_ref, wo_ref,
                bo_ref, final_ref, asum_acc, rvec_acc):
    i = pl.program_id(0)
    nb = pl.num_programs(0)
    q = _s2s_q(blih_ref, blhh_ref)
    s = s_ref[...]
    e = jnp.sum(s * q, axis=1, keepdims=True)
    gid = lax.broadcasted_iota(jnp.int32, (BN, NG), 1)
    mf = (bcol_ref[...] == gid).astype(jnp.float32)
    emax_pn = jnp.sum(mf * emax_ref[...], axis=1, keepdims=True)
    a = jnp.exp(e - emax_pn)
    gid2 = lax.broadcasted_iota(jnp.int32, (NG, BN), 0)
    mtf = (brow_ref[...] == gid2).astype(jnp.float32)
    asum_blk = jnp.dot(mtf, a, preferred_element_type=jnp.float32)
    rvec_blk = jnp.dot(mtf, a * s, preferred_element_type=jnp.float32)

    @pl.when(i == 0)
    def _():
        asum_acc[...] = jnp.zeros_like(asum_acc)
        rvec_acc[...] = jnp.zeros_like(rvec_acc)

    asum_acc[...] += asum_blk
    rvec_acc[...] += rvec_blk

    @pl.when(i == nb - 1)
    def _():
        asum = asum_acc[...]
        rinv = jnp.where(asum > 0, 1.0 / asum, 0.0)
        rvec = rvec_acc[...] * rinv
        wo = wo_ref[...]
        final_ref[...] = (
            jnp.dot(q, wo[0:DIM, :], preferred_element_type=jnp.float32)
            + jnp.dot(rvec, wo[DIM:2 * DIM, :], preferred_element_type=jnp.float32)
            + bo_ref[...]
        )


def _tc_s2s_b(s10, bcol, brow, emax, blihr, blhhr, wo, bor):
    return pl.pallas_call(
        _s2s_b_body,
        grid=(NP // BN,),
        in_specs=[
            pl.BlockSpec((BN, DIM), lambda i: (i, 0)),
            pl.BlockSpec((BN, 1), lambda i: (i, 0)),
            pl.BlockSpec((1, BN), lambda i: (0, i)),
            pl.BlockSpec((1, NG), lambda i: (0, 0)),
            pl.BlockSpec((1, 4 * DIM), lambda i: (0, 0)),
            pl.BlockSpec((1, 4 * DIM), lambda i: (0, 0)),
            pl.BlockSpec((2 * DIM, 2), lambda i: (0, 0)),
            pl.BlockSpec((1, 2), lambda i: (0, 0)),
        ],
        out_specs=pl.BlockSpec((NG, 2), lambda i: (0, 0)),
        out_shape=jax.ShapeDtypeStruct((NG, 2), jnp.float32),
        scratch_shapes=[
            pltpu.VMEM((NG, 1), jnp.float32),
            pltpu.VMEM((NG, DIM), jnp.float32),
        ],
    )(s10, bcol, brow, emax, blihr, blhhr, wo, bor)


def kernel(x, edge_index, edge_attr, stem_atmidx, jbond_atmidx, batch, W0, b0,
           Wn1, bn1, Wn2, bn2, Wroot, broot, Wih, Whh, bih, bhh, Ws1, bs1, Ws2,
           bs2, Wj1, bj1, Wj2, bj2, Wl_ih, Wl_hh, bl_ih, bl_hh, Wo, bo):
    f32 = jnp.float32
    i32 = jnp.int32
    src = edge_index[0]
    dst = edge_index[1]
    pe = EP - E
    srcp = jnp.concatenate([src, jnp.zeros((pe,), i32)])
    dstp = jnp.concatenate([dst, jnp.full((pe,), NP - 1, i32)])
    eap = jnp.concatenate([edge_attr, jnp.zeros((pe, 4), f32)], axis=0)
    xp = jnp.concatenate([x, jnp.zeros((NP - N, FEAT), f32)], axis=0)
    zpad_s = jnp.zeros((STEM_P - NSTEM,), i32)
    zpad_j = jnp.zeros((JB_P - NJB,), i32)
    ridx = jnp.concatenate([
        stem_atmidx, zpad_s,
        jbond_atmidx[:, 0], zpad_j,
        jbond_atmidx[:, 1], zpad_j,
    ])
    zeros_nd = jnp.zeros((NP, DIM), f32)
    ones_ch = jnp.ones((CH, DIM), f32)

    def _hl(v):
        hi = v.astype(jnp.bfloat16).astype(f32)
        return hi, v - hi

    tcat = jnp.transpose(Wn2.reshape(DIM, DIM, DIM), (1, 2, 0)).reshape(DIM, DIM * DIM)
    tcatb = jnp.concatenate([tcat, bn2.reshape(DIM, DIM)], axis=1)
    tb_hi, tb_lo = _hl(tcatb)
    tcat3 = jnp.concatenate([tb_hi, tb_lo, tb_hi], axis=0)
    w1_hi, w1_lo = _hl(Wn1)
    wn13 = jnp.concatenate([w1_hi, w1_lo, w1_hi], axis=0)
    sel = jnp.repeat(jnp.eye(DIM, dtype=f32), DIM, axis=0)
    b0r = b0.reshape(1, DIM)
    bn1r = bn1.reshape(1, DIM)
    brootr = broot.reshape(1, DIM)
    bihr = bih.reshape(1, 3 * DIM)
    bhhr = bhh.reshape(1, 3 * DIM)
    bs1r = bs1.reshape(1, DIM)
    bs2r = bs2.reshape(1, NOUT)
    bj1r = bj1.reshape(1, DIM)
    wj2r = Wj2.reshape(1, DIM)
    bj2r = bj2.reshape(1, 1)
    blihr = bl_ih.reshape(1, 4 * DIM)
    blhhr = bl_hh.reshape(1, 4 * DIM)
    bor = bo.reshape(1, 2)
    batchp = jnp.concatenate([batch, jnp.full((NP - N,), -1, i32)])
    bcol = batchp.reshape(NP, 1)
    brow = batchp.reshape(1, NP)

    degp = _sc_deg(dstp, ones_ch, zeros_nd)
    s, dinv = _tc_init(xp, W0, b0r, degp)
    for _ in range(6):
        xs = _sc_gather_edges(s, srcp)
        msg = _tc_msg(eap, xs, wn13, bn1r, tcat3, sel)
        aggp = _sc_scatter(msg, dstp, zeros_nd)
        s = _tc_gru(s, aggp, dinv, Wroot, brootr, Wih, bihr, Whh, bhhr)
    gath = _sc_gather_read(s, ridx)
    stem = _tc_stem(gath, Ws1, bs1r, Ws2, bs2r)
    jb = _tc_jb(gath, Wj1, bj1r, wj2r, bj2r)
    emax = _tc_s2s_a(s, bcol, blihr, blhhr)
    final = _tc_s2s_b(s, bcol, brow, emax, blihr, blhhr, Wo, bor)
    return final, stem[:NSTEM], jb[:NJB, 0]


# speedup vs baseline: 2.6767x; 1.0460x over previous
import functools

import jax
import jax.numpy as jnp
from jax import lax
from jax.experimental import pallas as pl
from jax.experimental.pallas import tpu as pltpu
from jax.experimental.pallas import tpu_sc as plsc

N = 10000
E = 160000
FEAT = 14
DIM = 32
NOUT = 105
NG = 256
NSTEM = 20000
NJB = 10000

NP = 10240
EP = 163840
NC = 2
NS = 16
NW = NC * NS
CH = 1024
ECHUNKS = EP // NW // CH
NSLC = NP // NS

STEM_P = 20480
JB_P = 10240
RLEN = STEM_P + 2 * JB_P
RCH = 1280
RCHUNKS = RLEN // NW // RCH

_mesh = plsc.VectorSubcoreMesh(core_axis_name="c", subcore_axis_name="s")
_sc_params = pltpu.CompilerParams(use_tc_tiling_on_sc=False)


def _leaky(v):
    return jnp.where(v >= 0, v, 0.01 * v)


def _sc_gather_body(nchunks, chunk, per_worker, idx_base, table, idx_hbm, out_hbm,
                    i0, i1, r0, r1, sg0, sg1, sw0, sw1):
    cid = lax.axis_index("c")
    sid = lax.axis_index("s")
    wid = sid * NC + cid
    base = wid * per_worker
    ib, rb = [i0, i1], [r0, r1]
    sg, sw = [sg0, sg1], [sw0, sw1]
    g, w = {}, {}
    pltpu.sync_copy(idx_hbm.at[pl.ds(idx_base + base, chunk)], i0)
    g[0] = pltpu.async_copy(table.at[i0], r0, sg0)
    for c in range(nchunks):
        cur, nxt = c % 2, (c + 1) % 2
        if c + 1 < nchunks:
            pltpu.sync_copy(idx_hbm.at[pl.ds(idx_base + base + (c + 1) * chunk, chunk)], ib[nxt])
            if c >= 1:
                w[c - 1].wait()
            g[c + 1] = pltpu.async_copy(table.at[ib[nxt]], rb[nxt], sg[nxt])
        elif c >= 1:
            w[c - 1].wait()
        g[c].wait()
        w[c] = pltpu.async_copy(rb[cur], out_hbm.at[pl.ds(base + c * chunk, chunk)], sw[cur])
    w[nchunks - 1].wait()


def _make_sc_gather(total, nchunks, chunk, idx_base=0):
    per_worker = total // NW
    return functools.partial(
        pl.kernel,
        out_type=jax.ShapeDtypeStruct((total, DIM), jnp.float32),
        mesh=_mesh,
        scratch_types=[
            pltpu.VMEM((chunk,), jnp.int32),
            pltpu.VMEM((chunk,), jnp.int32),
            pltpu.VMEM((chunk, DIM), jnp.float32),
            pltpu.VMEM((chunk, DIM), jnp.float32),
            pltpu.SemaphoreType.DMA,
            pltpu.SemaphoreType.DMA,
            pltpu.SemaphoreType.DMA,
            pltpu.SemaphoreType.DMA,
        ],
        compiler_params=_sc_params,
    )(functools.partial(_sc_gather_body, nchunks, chunk, per_worker, idx_base))


EH = EP // 2
HCH = 1280
HCHUNKS = EH // NW // HCH
_sc_gather_a = _make_sc_gather(EH, HCHUNKS, HCH, idx_base=0)
_sc_gather_b = _make_sc_gather(EH, HCHUNKS, HCH, idx_base=EH)
_sc_gather_read = _make_sc_gather(RLEN, RCHUNKS, RCH)


def _sc_scatter_body(dst_base, msg_hbm, dst_hbm, zeros_hbm, aggp_hbm, d0, d1, m0, m1,
                     acc, sm0, sm1):
    cid = lax.axis_index("c")
    sid = lax.axis_index("s")
    pltpu.sync_copy(zeros_hbm.at[pl.ds(sid * NSLC, NSLC)], acc.at[pl.ds(sid * NSLC, NSLC)])
    half = EH // NC
    base = cid * half + sid * (half // NS)
    db, mb, sm = [d0, d1], [m0, m1], [sm0, sm1]
    a = {}
    pltpu.sync_copy(dst_hbm.at[pl.ds(dst_base + base, HCH)], d0)
    a[0] = pltpu.async_copy(msg_hbm.at[pl.ds(base, HCH)], m0, sm0)
    plsc.subcore_barrier()
    for c in range(HCHUNKS):
        cur, nxt = c % 2, (c + 1) % 2
        if c + 1 < HCHUNKS:
            pltpu.sync_copy(dst_hbm.at[pl.ds(dst_base + base + (c + 1) * HCH, HCH)], db[nxt])
            a[c + 1] = pltpu.async_copy(msg_hbm.at[pl.ds(base + (c + 1) * HCH, HCH)], mb[nxt], sm[nxt])
        a[c].wait()
        pltpu.sync_copy(mb[cur], acc.at[db[cur]], add=True)
    plsc.subcore_barrier()
    pltpu.sync_copy(acc.at[pl.ds(sid * NSLC, NSLC)], aggp_hbm.at[cid, pl.ds(sid * NSLC, NSLC)])


def _make_sc_scatter(dst_base):
    return functools.partial(
        pl.kernel,
        out_type=jax.ShapeDtypeStruct((NC, NP, DIM), jnp.float32),
        mesh=_mesh,
        scratch_types=[
            pltpu.VMEM((HCH,), jnp.int32),
            pltpu.VMEM((HCH,), jnp.int32),
            pltpu.VMEM((HCH, DIM), jnp.float32),
            pltpu.VMEM((HCH, DIM), jnp.float32),
            pltpu.VMEM_SHARED((NP, DIM), jnp.float32),
            pltpu.SemaphoreType.DMA,
            pltpu.SemaphoreType.DMA,
        ],
        compiler_params=_sc_params,
    )(functools.partial(_sc_scatter_body, dst_base))


_sc_scatter_a = _make_sc_scatter(0)
_sc_scatter_b = _make_sc_scatter(EH)


def _sc_deg_body(dst_hbm, ones_hbm, zeros_hbm, degp_hbm, idxv, onesv, acc):
    cid = lax.axis_index("c")
    sid = lax.axis_index("s")
    pltpu.sync_copy(zeros_hbm.at[pl.ds(sid * NSLC, NSLC)], acc.at[pl.ds(sid * NSLC, NSLC)])
    pltpu.sync_copy(ones_hbm, onesv)
    plsc.subcore_barrier()
    half = EP // NC
    for ci in range(ECHUNKS):
        off = cid * half + sid * (half // NS) + ci * CH
        pltpu.sync_copy(dst_hbm.at[pl.ds(off, CH)], idxv)
        pltpu.sync_copy(onesv, acc.at[idxv], add=True)
    plsc.subcore_barrier()
    pltpu.sync_copy(acc.at[pl.ds(sid * NSLC, NSLC)], degp_hbm.at[cid, pl.ds(sid * NSLC, NSLC)])


_sc_deg = functools.partial(
    pl.kernel,
    out_type=jax.ShapeDtypeStruct((NC, NP, DIM), jnp.float32),
    mesh=_mesh,
    scratch_types=[
        pltpu.VMEM((CH,), jnp.int32),
        pltpu.VMEM((CH, DIM), jnp.float32),
        pltpu.VMEM_SHARED((NP, DIM), jnp.float32),
    ],
    compiler_params=_sc_params,
)(_sc_deg_body)


def _tc_init_body(xp_ref, w0_ref, b0_ref, degp_ref, s0_ref, dinv_ref):
    s0_ref[...] = _leaky(
        jnp.dot(xp_ref[...], w0_ref[...], preferred_element_type=jnp.float32) + b0_ref[...]
    )
    deg = jnp.maximum(degp_ref[0] + degp_ref[1], 1.0)
    dinv_ref[...] = 1.0 / deg


def _tc_init(xp, w0, b0r, degp):
    return pl.pallas_call(
        _tc_init_body,
        out_shape=(
            jax.ShapeDtypeStruct((NP, DIM), jnp.float32),
            jax.ShapeDtypeStruct((NP, DIM), jnp.float32),
        ),
    )(xp, w0, b0r, degp)


BE = 1024


def _split_hi_lo(v):
    hi = v.astype(jnp.bfloat16).astype(jnp.float32)
    return hi, v - hi


def _tc_msg_body(ea_ref, xs_ref, wn13_ref, bn1_ref, tcat3_ref, sel_ref, msg_ref):
    ea = ea_ref[...]
    ea_hi, ea_lo = _split_hi_lo(ea)
    ea3 = jnp.concatenate([ea_hi, ea_hi, ea_lo], axis=1)
    z = _leaky(
        jnp.dot(ea3, wn13_ref[...], preferred_element_type=jnp.float32) + bn1_ref[...]
    )
    xs = xs_ref[...]
    xs_hi, xs_lo = _split_hi_lo(xs)
    xs3 = jnp.concatenate([xs_hi, xs_hi, xs_lo], axis=1)
    h = jnp.dot(xs3, tcat3_ref[...], preferred_element_type=jnp.float32)
    p = h[:, :DIM * DIM] * jnp.tile(z, (1, DIM))
    msg_ref[...] = (
        jnp.dot(p, sel_ref[...], preferred_element_type=jnp.float32)
        + h[:, DIM * DIM:]
    )


def _make_tc_msg(base_blk):
    nblk = EH // BE

    def call(eap, xs, wn13, bn1r, tcat3, sel):
        return pl.pallas_call(
            _tc_msg_body,
            grid=(nblk,),
            in_specs=[
                pl.BlockSpec((BE, 4), lambda i: (i + base_blk, 0)),
                pl.BlockSpec((BE, DIM), lambda i: (i, 0)),
                pl.BlockSpec((12, DIM), lambda i: (0, 0)),
                pl.BlockSpec((1, DIM), lambda i: (0, 0)),
                pl.BlockSpec((3 * DIM, DIM * DIM + DIM), lambda i: (0, 0)),
                pl.BlockSpec((DIM * DIM, DIM), lambda i: (0, 0)),
            ],
            out_specs=pl.BlockSpec((BE, DIM), lambda i: (i, 0)),
            out_shape=jax.ShapeDtypeStruct((EH, DIM), jnp.float32),
        )(eap, xs, wn13, bn1r, tcat3, sel)

    return call


_tc_msg_a = _make_tc_msg(0)
_tc_msg_b = _make_tc_msg(EH // BE)


def _tc_gru_body(s_ref, aggp_ref, aggq_ref, dinv_ref, wroot_ref, broot_ref, wih_ref,
                 bih_ref, whh_ref, bhh_ref, out_ref):
    s = s_ref[...]
    agg = (aggp_ref[0] + aggp_ref[1] + aggq_ref[0] + aggq_ref[1]) * dinv_ref[...]
    m = _leaky(
        jnp.dot(s, wroot_ref[...], preferred_element_type=jnp.float32)
        + broot_ref[...] + agg
    )
    gi = jnp.dot(m, wih_ref[...], preferred_element_type=jnp.float32) + bih_ref[...]
    gh = jnp.dot(s, whh_ref[...], preferred_element_type=jnp.float32) + bhh_ref[...]
    r = jax.nn.sigmoid(gi[:, 0:DIM] + gh[:, 0:DIM])
    zz = jax.nn.sigmoid(gi[:, DIM:2 * DIM] + gh[:, DIM:2 * DIM])
    n = jnp.tanh(gi[:, 2 * DIM:3 * DIM] + r * gh[:, 2 * DIM:3 * DIM])
    out_ref[...] = (1.0 - zz) * n + zz * s


def _tc_gru(s, aggp, aggq, dinv, wroot, brootr, wih, bihr, whh, bhhr):
    return pl.pallas_call(
        _tc_gru_body,
        out_shape=jax.ShapeDtypeStruct((NP, DIM), jnp.float32),
    )(s, aggp, aggq, dinv, wroot, brootr, wih, bihr, whh, bhhr)


BS = 2048


def _tc_stem_body(g_ref, ws1_ref, bs1_ref, ws2_ref, bs2_ref, out_ref):
    act = _leaky(
        jnp.dot(g_ref[...], ws1_ref[...], preferred_element_type=jnp.float32) + bs1_ref[...]
    )
    out_ref[...] = jnp.dot(act, ws2_ref[...], preferred_element_type=jnp.float32) + bs2_ref[...]


def _tc_stem(gath, ws1, bs1r, ws2, bs2r):
    return pl.pallas_call(
        _tc_stem_body,
        grid=(STEM_P // BS,),
        in_specs=[
            pl.BlockSpec((BS, DIM), lambda i: (i, 0)),
            pl.BlockSpec((DIM, DIM), lambda i: (0, 0)),
            pl.BlockSpec((1, DIM), lambda i: (0, 0)),
            pl.BlockSpec((DIM, NOUT), lambda i: (0, 0)),
            pl.BlockSpec((1, NOUT), lambda i: (0, 0)),
        ],
        out_specs=pl.BlockSpec((BS, NOUT), lambda i: (i, 0)),
        out_shape=jax.ShapeDtypeStruct((STEM_P, NOUT), jnp.float32),
    )(gath, ws1, bs1r, ws2, bs2r)


def _tc_jb_body(g0_ref, g1_ref, wj1_ref, bj1_ref, wj2_ref, bj2_ref, out_ref):
    wj2 = wj2_ref[...]
    a0 = _leaky(
        jnp.dot(g0_ref[...], wj1_ref[...], preferred_element_type=jnp.float32) + bj1_ref[...]
    )
    v0 = jnp.sum(a0 * wj2, axis=1, keepdims=True) + bj2_ref[...]
    a1 = _leaky(
        jnp.dot(g1_ref[...], wj1_ref[...], preferred_element_type=jnp.float32) + bj1_ref[...]
    )
    v1 = jnp.sum(a1 * wj2, axis=1, keepdims=True) + bj2_ref[...]
    out_ref[...] = 0.5 * (v0 + v1)


def _tc_jb(gath, wj1, bj1r, wj2r, bj2r):
    blk0 = STEM_P // BS
    blk1 = (STEM_P + JB_P) // BS
    return pl.pallas_call(
        _tc_jb_body,
        grid=(JB_P // BS,),
        in_specs=[
            pl.BlockSpec((BS, DIM), lambda i: (blk0 + i, 0)),
            pl.BlockSpec((BS, DIM), lambda i: (blk1 + i, 0)),
            pl.BlockSpec((DIM, DIM), lambda i: (0, 0)),
            pl.BlockSpec((1, DIM), lambda i: (0, 0)),
            pl.BlockSpec((1, DIM), lambda i: (0, 0)),
            pl.BlockSpec((1, 1), lambda i: (0, 0)),
        ],
        out_specs=pl.BlockSpec((BS, 1), lambda i: (i, 0)),
        out_shape=jax.ShapeDtypeStruct((JB_P, 1), jnp.float32),
    )(gath, gath, wj1, bj1r, wj2r, bj2r)


BN = 2048


def _s2s_q(blih_ref, blhh_ref):
    gates = blih_ref[...] + blhh_ref[...]
    i_ = jax.nn.sigmoid(gates[:, 0:DIM])
    g_ = jnp.tanh(gates[:, 2 * DIM:3 * DIM])
    o_ = jax.nn.sigmoid(gates[:, 3 * DIM:4 * DIM])
    return o_ * jnp.tanh(i_ * g_)


def _s2s_a_body(s_ref, bcol_ref, blih_ref, blhh_ref, emax_ref, acc):
    i = pl.program_id(0)
    q = _s2s_q(blih_ref, blhh_ref)
    e = jnp.sum(s_ref[...] * q, axis=1, keepdims=True)
    gid = lax.broadcasted_iota(jnp.int32, (BN, NG), 1)
    m = bcol_ref[...] == gid
    bm = jnp.max(jnp.where(m, e, -1e30), axis=0, keepdims=True)

    @pl.when(i == 0)
    def _():
        acc[...] = bm

    @pl.when(i > 0)
    def _():
        acc[...] = jnp.maximum(acc[...], bm)

    emax_ref[...] = jnp.where(acc[...] > -1e29, acc[...], 0.0)


def _tc_s2s_a(s10, bcol, blihr, blhhr):
    return pl.pallas_call(
        _s2s_a_body,
        grid=(NP // BN,),
        in_specs=[
            pl.BlockSpec((BN, DIM), lambda i: (i, 0)),
            pl.BlockSpec((BN, 1), lambda i: (i, 0)),
            pl.BlockSpec((1, 4 * DIM), lambda i: (0, 0)),
            pl.BlockSpec((1, 4 * DIM), lambda i: (0, 0)),
        ],
        out_specs=pl.BlockSpec((1, NG), lambda i: (0, 0)),
        out_shape=jax.ShapeDtypeStruct((1, NG), jnp.float32),
        scratch_shapes=[pltpu.VMEM((1, NG), jnp.float32)],
    )(s10, bcol, blihr, blhhr)


def _s2s_b_body(s_ref, bcol_ref, brow_ref, emax_ref, blih_ref, blhh_ref, wo_ref,
                bo_ref, final_ref, asum_acc, rvec_acc):
    i = pl.program_id(0)
    nb = pl.num_programs(0)
    q = _s2s_q(blih_ref, blhh_ref)
    s = s_ref[...]
    e = jnp.sum(s * q, axis=1, keepdims=True)
    gid = lax.broadcasted_iota(jnp.int32, (BN, NG), 1)
    mf = (bcol_ref[...] == gid).astype(jnp.float32)
    emax_pn = jnp.sum(mf * emax_ref[...], axis=1, keepdims=True)
    a = jnp.exp(e - emax_pn)
    gid2 = lax.broadcasted_iota(jnp.int32, (NG, BN), 0)
    mtf = (brow_ref[...] == gid2).astype(jnp.float32)
    asum_blk = jnp.dot(mtf, a, preferred_element_type=jnp.float32)
    rvec_blk = jnp.dot(mtf, a * s, preferred_element_type=jnp.float32)

    @pl.when(i == 0)
    def _():
        asum_acc[...] = jnp.zeros_like(asum_acc)
        rvec_acc[...] = jnp.zeros_like(rvec_acc)

    asum_acc[...] += asum_blk
    rvec_acc[...] += rvec_blk

    @pl.when(i == nb - 1)
    def _():
        asum = asum_acc[...]
        rinv = jnp.where(asum > 0, 1.0 / asum, 0.0)
        rvec = rvec_acc[...] * rinv
        wo = wo_ref[...]
        final_ref[...] = (
            jnp.dot(q, wo[0:DIM, :], preferred_element_type=jnp.float32)
            + jnp.dot(rvec, wo[DIM:2 * DIM, :], preferred_element_type=jnp.float32)
            + bo_ref[...]
        )


def _tc_s2s_b(s10, bcol, brow, emax, blihr, blhhr, wo, bor):
    return pl.pallas_call(
        _s2s_b_body,
        grid=(NP // BN,),
        in_specs=[
            pl.BlockSpec((BN, DIM), lambda i: (i, 0)),
            pl.BlockSpec((BN, 1), lambda i: (i, 0)),
            pl.BlockSpec((1, BN), lambda i: (0, i)),
            pl.BlockSpec((1, NG), lambda i: (0, 0)),
            pl.BlockSpec((1, 4 * DIM), lambda i: (0, 0)),
            pl.BlockSpec((1, 4 * DIM), lambda i: (0, 0)),
            pl.BlockSpec((2 * DIM, 2), lambda i: (0, 0)),
            pl.BlockSpec((1, 2), lambda i: (0, 0)),
        ],
        out_specs=pl.BlockSpec((NG, 2), lambda i: (0, 0)),
        out_shape=jax.ShapeDtypeStruct((NG, 2), jnp.float32),
        scratch_shapes=[
            pltpu.VMEM((NG, 1), jnp.float32),
            pltpu.VMEM((NG, DIM), jnp.float32),
        ],
    )(s10, bcol, brow, emax, blihr, blhhr, wo, bor)


def kernel(x, edge_index, edge_attr, stem_atmidx, jbond_atmidx, batch, W0, b0,
           Wn1, bn1, Wn2, bn2, Wroot, broot, Wih, Whh, bih, bhh, Ws1, bs1, Ws2,
           bs2, Wj1, bj1, Wj2, bj2, Wl_ih, Wl_hh, bl_ih, bl_hh, Wo, bo):
    f32 = jnp.float32
    i32 = jnp.int32
    src = edge_index[0]
    dst = edge_index[1]
    pe = EP - E
    srcp = jnp.concatenate([src, jnp.zeros((pe,), i32)])
    dstp = jnp.concatenate([dst, jnp.full((pe,), NP - 1, i32)])
    eap = jnp.concatenate([edge_attr, jnp.zeros((pe, 4), f32)], axis=0)
    xp = jnp.concatenate([x, jnp.zeros((NP - N, FEAT), f32)], axis=0)
    zpad_s = jnp.zeros((STEM_P - NSTEM,), i32)
    zpad_j = jnp.zeros((JB_P - NJB,), i32)
    ridx = jnp.concatenate([
        stem_atmidx, zpad_s,
        jbond_atmidx[:, 0], zpad_j,
        jbond_atmidx[:, 1], zpad_j,
    ])
    zeros_nd = jnp.zeros((NP, DIM), f32)
    ones_ch = jnp.ones((CH, DIM), f32)

    def _hl(v):
        hi = v.astype(jnp.bfloat16).astype(f32)
        return hi, v - hi

    tcat = jnp.transpose(Wn2.reshape(DIM, DIM, DIM), (1, 2, 0)).reshape(DIM, DIM * DIM)
    tcatb = jnp.concatenate([tcat, bn2.reshape(DIM, DIM)], axis=1)
    tb_hi, tb_lo = _hl(tcatb)
    tcat3 = jnp.concatenate([tb_hi, tb_lo, tb_hi], axis=0)
    w1_hi, w1_lo = _hl(Wn1)
    wn13 = jnp.concatenate([w1_hi, w1_lo, w1_hi], axis=0)
    sel = jnp.repeat(jnp.eye(DIM, dtype=f32), DIM, axis=0)
    b0r = b0.reshape(1, DIM)
    bn1r = bn1.reshape(1, DIM)
    brootr = broot.reshape(1, DIM)
    bihr = bih.reshape(1, 3 * DIM)
    bhhr = bhh.reshape(1, 3 * DIM)
    bs1r = bs1.reshape(1, DIM)
    bs2r = bs2.reshape(1, NOUT)
    bj1r = bj1.reshape(1, DIM)
    wj2r = Wj2.reshape(1, DIM)
    bj2r = bj2.reshape(1, 1)
    blihr = bl_ih.reshape(1, 4 * DIM)
    blhhr = bl_hh.reshape(1, 4 * DIM)
    bor = bo.reshape(1, 2)
    batchp = jnp.concatenate([batch, jnp.full((NP - N,), -1, i32)])
    bcol = batchp.reshape(NP, 1)
    brow = batchp.reshape(1, NP)

    degp = _sc_deg(dstp, ones_ch, zeros_nd)
    s, dinv = _tc_init(xp, W0, b0r, degp)
    for _ in range(6):
        xs_a = _sc_gather_a(s, srcp)
        xs_b = _sc_gather_b(s, srcp)
        msg_a = _tc_msg_a(eap, xs_a, wn13, bn1r, tcat3, sel)
        aggp_a = _sc_scatter_a(msg_a, dstp, zeros_nd)
        msg_b = _tc_msg_b(eap, xs_b, wn13, bn1r, tcat3, sel)
        aggp_b = _sc_scatter_b(msg_b, dstp, zeros_nd)
        s = _tc_gru(s, aggp_a, aggp_b, dinv, Wroot, brootr, Wih, bihr, Whh, bhhr)
    gath = _sc_gather_read(s, ridx)
    stem = _tc_stem(gath, Ws1, bs1r, Ws2, bs2r)
    jb = _tc_jb(gath, Wj1, bj1r, wj2r, bj2r)
    emax = _tc_s2s_a(s, bcol, blihr, blhhr)
    final = _tc_s2s_b(s, bcol, brow, emax, blihr, blhhr, Wo, bor)
    return final, stem[:NSTEM], jb[:NJB, 0]
